# baseline (speedup 1.0000x reference)
import jax
import jax.numpy as jnp
from jax import lax
from jax.experimental import pallas as pl
from jax.experimental.pallas import tpu as pltpu
from jax.experimental.pallas import tpu_sc as plsc

B, P, D = 4, 8192, 768
N_ROWS = B * P
NC, NS, L = 2, 16, 16
NW = NC * NS
ROWS_PER_W = N_ROWS // NW
C = 16
NCHUNK = ROWS_PER_W // C
NSLOT = 4
LA = 2
VPR = D // L


def _sc_body(x_hbm, idx_hbm, pe_hbm, out_hbm, idx_v, xs, pes,
             sems_in, sems_out):
    wid = lax.axis_index("s") * NC + lax.axis_index("c")
    base0 = wid * ROWS_PER_W
    pltpu.sync_copy(idx_hbm.at[pl.ds(base0, ROWS_PER_W)], idx_v)

    def issue(j, b):
        pltpu.async_copy(
            pe_hbm.at[idx_v.at[pl.ds(j * C, C)]], pes[b], sems_in[b]
        )
        pltpu.async_copy(
            x_hbm.at[pl.ds(base0 + j * C, C)], xs[b], sems_in[b]
        )

    def drain_in(b):
        pltpu.make_async_copy(x_hbm.at[pl.ds(0, C)], xs[b], sems_in[b]).wait()
        pltpu.make_async_copy(x_hbm.at[pl.ds(0, C)], pes[b], sems_in[b]).wait()

    def drain_out(b):
        pltpu.make_async_copy(pes[b], out_hbm.at[pl.ds(0, C)], sems_out[b]).wait()

    def add_chunk(b):
        x_v, pe_v = xs[b], pes[b]

        @plsc.parallel_loop(0, C, step=1, unroll=2)
        def _row(r):
            for c in range(VPR):
                sl = pl.ds(c * L, L)
                pe_v[r, sl] = x_v[r, sl] + pe_v[r, sl]

    def store(j, b):
        pltpu.async_copy(pes[b], out_hbm.at[pl.ds(base0 + j * C, C)], sems_out[b])

    for s in range(LA):
        issue(s, s)

    def body(k, carry):
        for s in range(NSLOT):
            j = NSLOT * k + s
            t = (s + LA) % NSLOT

            @pl.when((j >= NSLOT - LA) & (j < NCHUNK - LA))
            def _():
                drain_out(t)

            @pl.when(j < NCHUNK - LA)
            def _():
                issue(j + LA, t)

            drain_in(s)
            add_chunk(s)
            store(j, s)
        return carry

    lax.fori_loop(0, NCHUNK // NSLOT, body, 0)
    for s in range(NSLOT):
        drain_out(s)


@jax.jit
def _pe_add(x2d, idx1d, pe2d):
    mesh = plsc.VectorSubcoreMesh(
        core_axis_name="c", subcore_axis_name="s", num_cores=NC, num_subcores=NS
    )

    def entry(x_hbm, idx_hbm, pe_hbm, out_hbm, idx_v,
              x0, x1, x2, x3, pe0, pe1, pe2, pe3,
              si0, si1, si2, si3, so0, so1, so2, so3):
        _sc_body(x_hbm, idx_hbm, pe_hbm, out_hbm, idx_v,
                 (x0, x1, x2, x3), (pe0, pe1, pe2, pe3),
                 (si0, si1, si2, si3), (so0, so1, so2, so3))

    f = pl.kernel(
        entry,
        out_type=jax.ShapeDtypeStruct((N_ROWS, D), jnp.float32),
        mesh=mesh,
        compiler_params=pltpu.CompilerParams(use_tc_tiling_on_sc=False),
        scratch_types=[pltpu.VMEM((ROWS_PER_W,), jnp.int32)]
        + [pltpu.VMEM((C, D), jnp.float32)] * (2 * NSLOT)
        + [pltpu.SemaphoreType.DMA] * (2 * NSLOT),
    )
    return f(x2d, idx1d, pe2d)


def kernel(x, indices, pe):
    out = _pe_add(
        x.reshape(N_ROWS, D), indices.reshape(N_ROWS), pe.reshape(P, D)
    )
    return out.reshape(B, P, D)


# speedup vs baseline: 2.4747x; 2.4747x over previous
import jax
import jax.numpy as jnp
from jax import lax
from jax.experimental import pallas as pl
from jax.experimental.pallas import tpu as pltpu
from jax.experimental.pallas import tpu_sc as plsc

B, P, D = 4, 8192, 768
N_ROWS = B * P
NC, NS, L = 2, 16, 16
NW = NC * NS
ROWS_PER_W = N_ROWS // NW
C = 8
NCHUNK = ROWS_PER_W // C
NSLOT = 8
LA = 4
VPR = D // L


def _sc_body(x_hbm, idx_hbm, pe_hbm, out_hbm, idx_v, xs, pes,
             sems_in, sems_out):
    wid = lax.axis_index("s") * NC + lax.axis_index("c")
    base0 = wid * ROWS_PER_W
    pltpu.sync_copy(idx_hbm.at[pl.ds(base0, ROWS_PER_W)], idx_v)

    def issue(j, b):
        pltpu.async_copy(
            pe_hbm.at[idx_v.at[pl.ds(j * C, C)]], pes[b], sems_in[b]
        )
        pltpu.async_copy(
            x_hbm.at[pl.ds(base0 + j * C, C)], xs[b], sems_in[b]
        )

    def drain_in(b):
        pltpu.make_async_copy(x_hbm.at[pl.ds(0, C)], xs[b], sems_in[b]).wait()
        pltpu.make_async_copy(x_hbm.at[pl.ds(0, C)], pes[b], sems_in[b]).wait()

    def drain_out(b):
        pltpu.make_async_copy(pes[b], out_hbm.at[pl.ds(0, C)], sems_out[b]).wait()

    def add_chunk(b):
        x_v, pe_v = xs[b], pes[b]

        @plsc.parallel_loop(0, C, step=1, unroll=2)
        def _row(r):
            for c in range(VPR):
                sl = pl.ds(c * L, L)
                pe_v[r, sl] = x_v[r, sl] + pe_v[r, sl]

    def store(j, b):
        pltpu.async_copy(pes[b], out_hbm.at[pl.ds(base0 + j * C, C)], sems_out[b])

    for s in range(LA):
        issue(s, s)

    def body(k, carry):
        for s in range(NSLOT):
            j = NSLOT * k + s
            t = (s + LA) % NSLOT

            @pl.when((j >= NSLOT - LA) & (j < NCHUNK - LA))
            def _():
                drain_out(t)

            @pl.when(j < NCHUNK - LA)
            def _():
                issue(j + LA, t)

            drain_in(s)
            add_chunk(s)
            store(j, s)
        return carry

    lax.fori_loop(0, NCHUNK // NSLOT, body, 0)
    for s in range(NSLOT):
        drain_out(s)


@jax.jit
def _pe_add(x2d, idx1d, pe2d):
    mesh = plsc.VectorSubcoreMesh(
        core_axis_name="c", subcore_axis_name="s", num_cores=NC, num_subcores=NS
    )

    def entry(x_hbm, idx_hbm, pe_hbm, out_hbm, idx_v, *rest):
        xs = rest[:NSLOT]
        pes = rest[NSLOT:2 * NSLOT]
        sems_in = rest[2 * NSLOT:3 * NSLOT]
        sems_out = rest[3 * NSLOT:4 * NSLOT]
        _sc_body(x_hbm, idx_hbm, pe_hbm, out_hbm, idx_v,
                 xs, pes, sems_in, sems_out)

    f = pl.kernel(
        entry,
        out_type=jax.ShapeDtypeStruct((N_ROWS, D), jnp.float32),
        mesh=mesh,
        scratch_types=[pltpu.VMEM((ROWS_PER_W,), jnp.int32)]
        + [pltpu.VMEM((C, D), jnp.float32)] * (2 * NSLOT)
        + [pltpu.SemaphoreType.DMA] * (2 * NSLOT),
    )
    return f(x2d, idx1d, pe2d)


def kernel(x, indices, pe):
    out = _pe_add(
        x.reshape(N_ROWS, D), indices.reshape(N_ROWS), pe.reshape(P, D)
    )
    return out.reshape(B, P, D)


# speedup vs baseline: 2.5927x; 1.0477x over previous
import jax
import jax.numpy as jnp
from jax import lax
from jax.experimental import pallas as pl
from jax.experimental.pallas import tpu as pltpu
from jax.experimental.pallas import tpu_sc as plsc

B, P, D = 4, 8192, 768
N_ROWS = B * P
NC, NS, L = 2, 16, 16
NW = NC * NS
ROWS_PER_W = N_ROWS // NW
C = 16
NCHUNK = ROWS_PER_W // C
NSLOT = 5
LA = 3
VPR = D // L


def _sc_body(x_hbm, idx_hbm, pe_hbm, out_hbm, idx_v, xs, pes,
             sems_in, sems_out):
    wid = lax.axis_index("s") * NC + lax.axis_index("c")
    base0 = wid * ROWS_PER_W
    pltpu.sync_copy(idx_hbm.at[pl.ds(base0, ROWS_PER_W)], idx_v)

    def issue(j, b):
        pltpu.async_copy(
            pe_hbm.at[idx_v.at[pl.ds(j * C, C)]], pes[b], sems_in[b]
        )
        pltpu.async_copy(
            x_hbm.at[pl.ds(base0 + j * C, C)], xs[b], sems_in[b]
        )

    def drain_in(b):
        pltpu.make_async_copy(x_hbm.at[pl.ds(0, C)], xs[b], sems_in[b]).wait()
        pltpu.make_async_copy(x_hbm.at[pl.ds(0, C)], pes[b], sems_in[b]).wait()

    def drain_out(b):
        pltpu.make_async_copy(pes[b], out_hbm.at[pl.ds(0, C)], sems_out[b]).wait()

    def add_chunk(b):
        x_v, pe_v = xs[b], pes[b]

        @plsc.parallel_loop(0, C, step=1, unroll=2)
        def _row(r):
            for c in range(VPR):
                sl = pl.ds(c * L, L)
                pe_v[r, sl] = x_v[r, sl] + pe_v[r, sl]

    def store(j, b):
        pltpu.async_copy(pes[b], out_hbm.at[pl.ds(base0 + j * C, C)], sems_out[b])

    for s in range(LA):
        issue(s, s)

    NBODY = (NCHUNK // NSLOT) * NSLOT

    def step(j, s):
        t = (s + LA) % NSLOT

        @pl.when((j >= NSLOT - LA) & (j < NCHUNK - LA))
        def _():
            drain_out(t)

        @pl.when(j < NCHUNK - LA)
        def _():
            issue(j + LA, t)

        drain_in(s)
        add_chunk(s)
        store(j, s)

    def body(k, carry):
        for s in range(NSLOT):
            step(NSLOT * k + s, s)
        return carry

    lax.fori_loop(0, NCHUNK // NSLOT, body, 0)
    for j in range(NBODY, NCHUNK):
        step(j, j % NSLOT)
    for s in range(NSLOT):
        drain_out(s)


@jax.jit
def _pe_add(x2d, idx1d, pe2d):
    mesh = plsc.VectorSubcoreMesh(
        core_axis_name="c", subcore_axis_name="s", num_cores=NC, num_subcores=NS
    )

    def entry(x_hbm, idx_hbm, pe_hbm, out_hbm, idx_v, *rest):
        xs = rest[:NSLOT]
        pes = rest[NSLOT:2 * NSLOT]
        sems_in = rest[2 * NSLOT:3 * NSLOT]
        sems_out = rest[3 * NSLOT:4 * NSLOT]
        _sc_body(x_hbm, idx_hbm, pe_hbm, out_hbm, idx_v,
                 xs, pes, sems_in, sems_out)

    f = pl.kernel(
        entry,
        out_type=jax.ShapeDtypeStruct((N_ROWS, D), jnp.float32),
        mesh=mesh,
        scratch_types=[pltpu.VMEM((ROWS_PER_W,), jnp.int32)]
        + [pltpu.VMEM((C, D), jnp.float32)] * (2 * NSLOT)
        + [pltpu.SemaphoreType.DMA] * (2 * NSLOT),
    )
    return f(x2d, idx1d, pe2d)


def kernel(x, indices, pe):
    out = _pe_add(
        x.reshape(N_ROWS, D), indices.reshape(N_ROWS), pe.reshape(P, D)
    )
    return out.reshape(B, P, D)


# speedup vs baseline: 2.5964x; 1.0014x over previous
import jax
import jax.numpy as jnp
from jax import lax
from jax.experimental import pallas as pl
from jax.experimental.pallas import tpu as pltpu
from jax.experimental.pallas import tpu_sc as plsc

B, P, D = 4, 8192, 768
N_ROWS = B * P
NC, NS, L = 2, 16, 16
NW = NC * NS
ROWS_PER_W = N_ROWS // NW
C = 16
NCHUNK = ROWS_PER_W // C
NSLOT = 4
LA = 2
VPR = D // L


def _sc_body(x_hbm, idx_hbm, pe_hbm, out_hbm, idx_v, xs, pes,
             sems_in, sems_out):
    wid = lax.axis_index("s") * NC + lax.axis_index("c")
    base0 = wid * ROWS_PER_W
    pltpu.sync_copy(idx_hbm.at[pl.ds(base0, ROWS_PER_W)], idx_v)

    def issue(j, b):
        pltpu.async_copy(
            pe_hbm.at[idx_v.at[pl.ds(j * C, C)]], pes[b], sems_in[b]
        )
        pltpu.async_copy(
            x_hbm.at[pl.ds(base0 + j * C, C)], xs[b], sems_in[b]
        )

    def drain_in(b):
        pltpu.make_async_copy(x_hbm.at[pl.ds(0, C)], xs[b], sems_in[b]).wait()
        pltpu.make_async_copy(x_hbm.at[pl.ds(0, C)], pes[b], sems_in[b]).wait()

    def drain_out(b):
        pltpu.make_async_copy(pes[b], out_hbm.at[pl.ds(0, C)], sems_out[b]).wait()

    def add_chunk(b):
        x_v, pe_v = xs[b], pes[b]

        @plsc.parallel_loop(0, C, step=1, unroll=2)
        def _row(r):
            for c in range(VPR):
                sl = pl.ds(c * L, L)
                pe_v[r, sl] = x_v[r, sl] + pe_v[r, sl]

    def store(j, b):
        pltpu.async_copy(pes[b], out_hbm.at[pl.ds(base0 + j * C, C)], sems_out[b])

    def step(j, s, head, tail):
        t = (s + LA) % NSLOT
        if not (head or tail):
            drain_out(t)
        if not tail:
            issue(j + LA, t)
        drain_in(s)
        add_chunk(s)
        store(j, s)

    for s in range(LA):
        issue(s, s)
    for j in range(NSLOT - LA):
        step(j, j % NSLOT, True, False)
    for j in range(NSLOT - LA, NSLOT):
        step(j, j % NSLOT, False, False)

    NSTEADY = (NCHUNK - LA - NSLOT) // NSLOT

    def body(k, carry):
        for s in range(NSLOT):
            step(NSLOT * (k + 1) + s, s, False, False)
        return carry

    lax.fori_loop(0, NSTEADY, body, 0)

    for j in range(NSLOT * (NSTEADY + 1), NCHUNK):
        tail = j >= NCHUNK - LA
        step(j, j % NSLOT, False, tail)
    for s in range(NSLOT):
        drain_out(s)


@jax.jit
def _pe_add(x2d, idx1d, pe2d):
    mesh = plsc.VectorSubcoreMesh(
        core_axis_name="c", subcore_axis_name="s", num_cores=NC, num_subcores=NS
    )

    def entry(x_hbm, idx_hbm, pe_hbm, out_hbm, idx_v,
              x0, x1, x2, x3, pe0, pe1, pe2, pe3,
              si0, si1, si2, si3, so0, so1, so2, so3):
        _sc_body(x_hbm, idx_hbm, pe_hbm, out_hbm, idx_v,
                 (x0, x1, x2, x3), (pe0, pe1, pe2, pe3),
                 (si0, si1, si2, si3), (so0, so1, so2, so3))

    f = pl.kernel(
        entry,
        out_type=jax.ShapeDtypeStruct((N_ROWS, D), jnp.float32),
        mesh=mesh,
        scratch_types=[pltpu.VMEM((ROWS_PER_W,), jnp.int32)]
        + [pltpu.VMEM((C, D), jnp.float32)] * (2 * NSLOT)
        + [pltpu.SemaphoreType.DMA] * (2 * NSLOT),
    )
    return f(x2d, idx1d, pe2d)


def kernel(x, indices, pe):
    out = _pe_add(
        x.reshape(N_ROWS, D), indices.reshape(N_ROWS), pe.reshape(P, D)
    )
    return out.reshape(B, P, D)


# speedup vs baseline: 2.7173x; 1.0466x over previous
import jax
import jax.numpy as jnp
from jax import lax
from jax.experimental import pallas as pl
from jax.experimental.pallas import tpu as pltpu
from jax.experimental.pallas import tpu_sc as plsc

B, P, D = 4, 8192, 768
N_ROWS = B * P
NC, NS, L = 2, 16, 16
NW = NC * NS
ROWS_PER_W = N_ROWS // NW
C = 16
NCHUNK = ROWS_PER_W // C
NSLOT = 4
LA = 2
VPR = D // L


def _sc_body(x_hbm, idx_hbm, pe_hbm, out_hbm, idx_v, xs, pes,
             sems_in, sems_out):
    wid = lax.axis_index("s") * NC + lax.axis_index("c")
    base0 = wid * ROWS_PER_W
    pltpu.sync_copy(idx_hbm.at[pl.ds(base0, ROWS_PER_W)], idx_v)

    def issue(j, b):
        pltpu.async_copy(
            pe_hbm.at[idx_v.at[pl.ds(j * C, C)]], pes[b], sems_in[b]
        )
        pltpu.async_copy(
            x_hbm.at[pl.ds(base0 + j * C, C)], xs[b], sems_in[b]
        )

    def drain_in(b):
        pltpu.make_async_copy(x_hbm.at[pl.ds(0, C)], xs[b], sems_in[b]).wait()
        pltpu.make_async_copy(x_hbm.at[pl.ds(0, C)], pes[b], sems_in[b]).wait()

    def drain_out(b):
        pltpu.make_async_copy(pes[b], out_hbm.at[pl.ds(0, C)], sems_out[b]).wait()

    def add_chunk(b):
        x_v, pe_v = xs[b], pes[b]

        @plsc.parallel_loop(0, C, step=1, unroll=2)
        def _row(r):
            for c in range(VPR):
                sl = pl.ds(c * L, L)
                pe_v[r, sl] = x_v[r, sl] + pe_v[r, sl]

    def store(j, b):
        pltpu.async_copy(pes[b], out_hbm.at[pl.ds(base0 + j * C, C)], sems_out[b])

    for s in range(LA):
        issue(s, s)

    def body(k, carry):
        for s in range(NSLOT):
            j = NSLOT * k + s
            t = (s + LA) % NSLOT

            @pl.when((j >= NSLOT - LA) & (j < NCHUNK - LA))
            def _():
                drain_out(t)

            @pl.when(j < NCHUNK - LA)
            def _():
                issue(j + LA, t)

            drain_in(s)
            add_chunk(s)
            store(j, s)
        return carry

    lax.fori_loop(0, NCHUNK // NSLOT, body, 0)
    for s in range(NSLOT):
        drain_out(s)


@jax.jit
def _pe_add(x2d, idx1d, pe2d):
    mesh = plsc.VectorSubcoreMesh(
        core_axis_name="c", subcore_axis_name="s", num_cores=NC, num_subcores=NS
    )

    def entry(x_hbm, idx_hbm, pe_hbm, out_hbm, idx_v,
              x0, x1, x2, x3, pe0, pe1, pe2, pe3,
              si0, si1, si2, si3, so0, so1, so2, so3):
        _sc_body(x_hbm, idx_hbm, pe_hbm, out_hbm, idx_v,
                 (x0, x1, x2, x3), (pe0, pe1, pe2, pe3),
                 (si0, si1, si2, si3), (so0, so1, so2, so3))

    f = pl.kernel(
        entry,
        out_type=jax.ShapeDtypeStruct((N_ROWS, D), jnp.float32),
        mesh=mesh,
        scratch_types=[pltpu.VMEM((ROWS_PER_W,), jnp.int32)]
        + [pltpu.VMEM((C, D), jnp.float32)] * (2 * NSLOT)
        + [pltpu.SemaphoreType.DMA] * (2 * NSLOT),
    )
    return f(x2d, idx1d, pe2d)


def kernel(x, indices, pe):
    out = _pe_add(
        x.reshape(N_ROWS, D), indices.reshape(N_ROWS), pe.reshape(P, D)
    )
    return out.reshape(B, P, D)
